# baseline (speedup 1.0000x reference)
import functools

import jax
import jax.numpy as jnp
from jax import lax
from jax.experimental import pallas as pl
from jax.experimental.pallas import tpu as pltpu
from jax.experimental.pallas import tpu_sc as plsc

NC = 2
NS = 16
NW = NC * NS
L = 16
D = 32

S = 16
DESC = 80
NDESC = 10


def _rsqrt_nr(s):
    i = plsc.bitcast(s, jnp.int32)
    i = jnp.int32(0x5F3759DF) - lax.shift_right_logical(i, 1)
    y = plsc.bitcast(i, jnp.float32)
    for _ in range(3):
        y = y * (1.5 - 0.5 * s * y * y)
    return y


@functools.cache
def _build(b, l):
    rows_per_w = b // NW
    steps = rows_per_w // S
    assert b % NW == 0 and rows_per_w % S == 0 and steps % 2 == 0

    mesh = plsc.VectorSubcoreMesh(
        core_axis_name="c", subcore_axis_name="s", num_cores=NC, num_subcores=NS
    )

    @functools.partial(
        pl.kernel,
        out_type=jax.ShapeDtypeStruct((l, D // 8, b // 128, 8, 128), jnp.float32),
        mesh=mesh,
        compiler_params=pltpu.CompilerParams(
            needs_layout_passes=False, use_tc_tiling_on_sc=False
        ),
        scratch_types=[
            pltpu.VMEM((2, S * l), jnp.int32),
            pltpu.VMEM((2, S * l, D), jnp.float32),
            pltpu.VMEM((2, l, D // 8, 8, S), jnp.float32),
            pltpu.SemaphoreType.DMA,
            pltpu.SemaphoreType.DMA,
            pltpu.SemaphoreType.DMA,
            pltpu.SemaphoreType.DMA,
        ],
    )
    def gather_norm(x_hbm, table_hbm, out_hbm, idx_v, rows_v, blk_v, g0, g1, w0, w1):
        wid = lax.axis_index("s") * NC + lax.axis_index("c")
        base = wid * rows_per_w
        lane = lax.iota(jnp.int32, L)
        gsems = (g0, g1)
        wsems = (w0, w1)

        def fire(t, p):
            row0 = base + t * S
            pltpu.sync_copy(x_hbm.at[pl.ds(row0 * l, S * l)], idx_v.at[p])
            for i in range(NDESC):
                pltpu.async_copy(
                    table_hbm.at[idx_v.at[p, pl.ds(i * DESC, DESC)]],
                    rows_v.at[p, pl.ds(i * DESC, DESC)], gsems[p],
                )

        def drain_gather(t, p):
            for i in range(NDESC):
                pltpu.make_async_copy(
                    table_hbm.at[idx_v.at[p, pl.ds(i * DESC, DESC)]],
                    rows_v.at[p, pl.ds(i * DESC, DESC)], gsems[p],
                ).wait()

        def norm(t, p):
            lane_l = lane * l

            @plsc.parallel_loop(0, l, 1, unroll=4)
            def body(j):
                rv = lane_l + j
                cols = [
                    plsc.load_gather(
                        rows_v,
                        [jnp.full((L,), p, jnp.int32), rv,
                         jnp.full((L,), d, jnp.int32)],
                    )
                    for d in range(D)
                ]
                sq = [c * c for c in cols]
                while len(sq) > 1:
                    sq = [a + b for a, b in zip(sq[::2], sq[1::2])]
                s = sq[0]
                y = _rsqrt_nr(s)
                for d, c in enumerate(cols):
                    blk_v[p, j, d // 8, d % 8, pl.ds(0, S)] = c * y

        def _write_dst(t):
            row0 = base + t * S
            return out_hbm.at[:, :, row0 // 128, :, pl.ds(row0 % 128, S)]

        def write(t, p):
            pltpu.async_copy(blk_v.at[p], _write_dst(t), wsems[p])

        def drain_write(t, p):
            pltpu.make_async_copy(blk_v.at[p], _write_dst(t), wsems[p]).wait()

        fire(0, 0)

        def pair(t2, carry):
            for p in (0, 1):
                t = t2 * 2 + p
                drain_gather(t, p)

                @pl.when(t + 1 < steps)
                def _():
                    fire(t + 1, 1 - p)

                @pl.when(t >= 2)
                def _():
                    drain_write(t - 2, p)

                if True:
                    norm(t, p)
                write(t, p)
            return carry

        lax.fori_loop(0, steps // 2, pair, 0)
        drain_write(steps - 2, 0)
        drain_write(steps - 1, 1)

    return gather_norm


def kernel(x, table):
    b, l = x.shape
    n, d = table.shape
    xf, tf = lax.optimization_barrier(
        (x.astype(jnp.int32).reshape(-1), table.reshape(n * d // 128, 128))
    )
    out5 = _build(b, l)(xf, tf.reshape(n, d))
    out_t = out5.transpose(0, 1, 3, 2, 4).reshape(l, D, b)
    return out_t.transpose(2, 0, 1)


# speedup vs baseline: 1.0553x; 1.0553x over previous
import functools

import jax
import jax.numpy as jnp
from jax import lax
from jax.experimental import pallas as pl
from jax.experimental.pallas import tpu as pltpu
from jax.experimental.pallas import tpu_sc as plsc

NC = 2
NS = 16
NW = NC * NS
L = 16
D = 32

S = 16
DESC = 80
NDESC = 10


def _rsqrt_nr(s):
    i = plsc.bitcast(s, jnp.int32)
    i = jnp.int32(0x5F3759DF) - lax.shift_right_logical(i, 1)
    y = plsc.bitcast(i, jnp.float32)
    for _ in range(3):
        y = y * (1.5 - 0.5 * s * y * y)
    return y


@functools.cache
def _build(b, l):
    rows_per_w = b // NW
    steps = rows_per_w // S
    assert b % NW == 0 and rows_per_w % S == 0 and steps % 2 == 0

    mesh = plsc.VectorSubcoreMesh(
        core_axis_name="c", subcore_axis_name="s", num_cores=NC, num_subcores=NS
    )

    @functools.partial(
        pl.kernel,
        out_type=jax.ShapeDtypeStruct((l, D // 8, b // 128, 8, 128), jnp.float32),
        mesh=mesh,
        compiler_params=pltpu.CompilerParams(
            needs_layout_passes=False, use_tc_tiling_on_sc=False
        ),
        scratch_types=[
            pltpu.VMEM((2, S * l), jnp.int32),
            pltpu.VMEM((2, S * l, D), jnp.float32),
            pltpu.VMEM((2, l, D // 8, 8, S), jnp.float32),
            pltpu.SemaphoreType.DMA,
            pltpu.SemaphoreType.DMA,
            pltpu.SemaphoreType.DMA,
            pltpu.SemaphoreType.DMA,
        ],
    )
    def gather_norm(x_hbm, table_hbm, out_hbm, idx_v, rows_v, blk_v, g0, g1, w0, w1):
        wid = lax.axis_index("s") * NC + lax.axis_index("c")
        base = wid * rows_per_w
        lane = lax.iota(jnp.int32, L)
        gsems = (g0, g1)
        wsems = (w0, w1)

        def fire(t, p):
            row0 = base + t * S
            pltpu.sync_copy(x_hbm.at[pl.ds(row0 * l, S * l)], idx_v.at[p])
            for i in range(NDESC):
                pltpu.async_copy(
                    table_hbm.at[idx_v.at[p, pl.ds(i * DESC, DESC)]],
                    rows_v.at[p, pl.ds(i * DESC, DESC)], gsems[p],
                )

        def drain_gather(t, p):
            for i in range(NDESC):
                pltpu.make_async_copy(
                    table_hbm.at[idx_v.at[p, pl.ds(i * DESC, DESC)]],
                    rows_v.at[p, pl.ds(i * DESC, DESC)], gsems[p],
                ).wait()

        def norm(t, p):
            lane_l = lane * l

            def one(j):
                rv = lane_l + j
                cols = [
                    plsc.load_gather(
                        rows_v,
                        [jnp.full((L,), p, jnp.int32), rv,
                         jnp.full((L,), d, jnp.int32)],
                    )
                    for d in range(D)
                ]
                sq = [c * c for c in cols]
                while len(sq) > 1:
                    sq = [a + b for a, b in zip(sq[::2], sq[1::2])]
                y = _rsqrt_nr(sq[0])
                for d, c in enumerate(cols):
                    blk_v[p, j, d // 8, d % 8, pl.ds(0, S)] = c * y

            def body(j2, carry):
                one(j2 * 2)
                one(j2 * 2 + 1)
                return carry

            lax.fori_loop(0, l // 2, body, 0)

        def _write_dst(t):
            row0 = base + t * S
            return out_hbm.at[:, :, row0 // 128, :, pl.ds(row0 % 128, S)]

        def write(t, p):
            pltpu.async_copy(blk_v.at[p], _write_dst(t), wsems[p])

        def drain_write(t, p):
            pltpu.make_async_copy(blk_v.at[p], _write_dst(t), wsems[p]).wait()

        fire(0, 0)

        def pair(t2, carry):
            for p in (0, 1):
                t = t2 * 2 + p
                drain_gather(t, p)

                @pl.when(t + 1 < steps)
                def _():
                    fire(t + 1, 1 - p)

                @pl.when(t >= 2)
                def _():
                    drain_write(t - 2, p)

                if True:
                    norm(t, p)
                write(t, p)
            return carry

        lax.fori_loop(0, steps // 2, pair, 0)
        drain_write(steps - 2, 0)
        drain_write(steps - 1, 1)

    return gather_norm


def kernel(x, table):
    b, l = x.shape
    n, d = table.shape
    xf, tf = lax.optimization_barrier(
        (x.astype(jnp.int32).reshape(-1), table.reshape(n * d // 128, 128))
    )
    out5 = _build(b, l)(xf, tf.reshape(n, d))
    out_t = out5.transpose(0, 1, 3, 2, 4).reshape(l, D, b)
    return out_t.transpose(2, 0, 1)


# speedup vs baseline: 1.0553x; 1.0000x over previous
import functools

import jax
import jax.numpy as jnp
from jax import lax
from jax.experimental import pallas as pl
from jax.experimental.pallas import tpu as pltpu
from jax.experimental.pallas import tpu_sc as plsc

NC = 2
NS = 16
NW = NC * NS
L = 16
D = 32

S = 16
DESC = 80
NDESC = 10


def _rsqrt_nr(s):
    i = plsc.bitcast(s, jnp.int32)
    i = jnp.int32(0x5F3759DF) - lax.shift_right_logical(i, 1)
    y = plsc.bitcast(i, jnp.float32)
    for _ in range(3):
        y = y * (1.5 - 0.5 * s * y * y)
    return y


@functools.cache
def _build(b, l):
    rows_per_w = b // NW
    steps = rows_per_w // S
    assert b % NW == 0 and rows_per_w % S == 0 and steps % 2 == 0

    mesh = plsc.VectorSubcoreMesh(
        core_axis_name="c", subcore_axis_name="s", num_cores=NC, num_subcores=NS
    )

    @functools.partial(
        pl.kernel,
        out_type=jax.ShapeDtypeStruct((l, D // 8, b // 128, 8, 128), jnp.float32),
        mesh=mesh,
        compiler_params=pltpu.CompilerParams(
            needs_layout_passes=False, use_tc_tiling_on_sc=False
        ),
        scratch_types=[
            pltpu.VMEM((2, S * l), jnp.int32),
            pltpu.VMEM((2, S * l, D), jnp.float32),
            pltpu.VMEM((2, l, D // 8, 8, S), jnp.float32),
            pltpu.SemaphoreType.DMA,
            pltpu.SemaphoreType.DMA,
            pltpu.SemaphoreType.DMA,
            pltpu.SemaphoreType.DMA,
        ],
    )
    def gather_norm(x_hbm, table_hbm, out_hbm, idx_v, rows_v, blk_v, g0, g1, w0, w1):
        wid = lax.axis_index("s") * NC + lax.axis_index("c")
        base = wid * rows_per_w
        lane = lax.iota(jnp.int32, L)
        gsems = (g0, g1)
        wsems = (w0, w1)

        def fire(t, p):
            row0 = base + t * S
            pltpu.sync_copy(x_hbm.at[pl.ds(row0 * l, S * l)], idx_v.at[p])
            for i in range(NDESC):
                pltpu.async_copy(
                    table_hbm.at[idx_v.at[p, pl.ds(i * DESC, DESC)]],
                    rows_v.at[p, pl.ds(i * DESC, DESC)], gsems[p],
                )

        def drain_gather(t, p):
            for i in range(NDESC):
                pltpu.make_async_copy(
                    table_hbm.at[idx_v.at[p, pl.ds(i * DESC, DESC)]],
                    rows_v.at[p, pl.ds(i * DESC, DESC)], gsems[p],
                ).wait()

        def norm(t, p):
            lane_l = lane * l

            rows_p = rows_v.at[p]

            def one(j):
                rv = lane_l + j
                cols = [
                    plsc.load_gather(rows_p, [rv, jnp.full((L,), d, jnp.int32)])
                    for d in range(D)
                ]
                sq = [c * c for c in cols]
                while len(sq) > 1:
                    sq = [a + b for a, b in zip(sq[::2], sq[1::2])]
                y = _rsqrt_nr(sq[0])
                for d, c in enumerate(cols):
                    blk_v[p, j, d // 8, d % 8, pl.ds(0, S)] = c * y

            def body(j2, carry):
                one(j2 * 2)
                one(j2 * 2 + 1)
                return carry

            lax.fori_loop(0, l // 2, body, 0)

        def _write_dst(t):
            row0 = base + t * S
            return out_hbm.at[:, :, row0 // 128, :, pl.ds(row0 % 128, S)]

        def write(t, p):
            pltpu.async_copy(blk_v.at[p], _write_dst(t), wsems[p])

        def drain_write(t, p):
            pltpu.make_async_copy(blk_v.at[p], _write_dst(t), wsems[p]).wait()

        fire(0, 0)

        def pair(t2, carry):
            for p in (0, 1):
                t = t2 * 2 + p
                drain_gather(t, p)

                @pl.when(t + 1 < steps)
                def _():
                    fire(t + 1, 1 - p)

                @pl.when(t >= 2)
                def _():
                    drain_write(t - 2, p)

                if True:
                    norm(t, p)
                write(t, p)
            return carry

        lax.fori_loop(0, steps // 2, pair, 0)
        drain_write(steps - 2, 0)
        drain_write(steps - 1, 1)

    return gather_norm


def kernel(x, table):
    b, l = x.shape
    n, d = table.shape
    xf, tf = lax.optimization_barrier(
        (x.astype(jnp.int32).reshape(-1), table.reshape(n * d // 128, 128))
    )
    out5 = _build(b, l)(xf, tf.reshape(n, d))
    out_t = out5.transpose(0, 1, 3, 2, 4).reshape(l, D, b)
    return out_t.transpose(2, 0, 1)


# speedup vs baseline: 1.4800x; 1.4024x over previous
import functools

import jax
import jax.numpy as jnp
from jax import lax
from jax.experimental import pallas as pl
from jax.experimental.pallas import tpu as pltpu
from jax.experimental.pallas import tpu_sc as plsc

NC = 2
NS = 16
NW = NC * NS
L = 16
D = 32

S = 16
DESC = 80
NDESC = 10


def _rsqrt_nr(s):
    i = plsc.bitcast(s, jnp.int32)
    i = jnp.int32(0x5F3759DF) - lax.shift_right_logical(i, 1)
    y = plsc.bitcast(i, jnp.float32)
    for _ in range(3):
        y = y * (1.5 - 0.5 * s * y * y)
    return y


@functools.cache
def _build(b, l):
    rows_per_w = b // NW
    steps = rows_per_w // S
    assert b % NW == 0 and rows_per_w % S == 0 and steps % 2 == 0

    mesh = plsc.VectorSubcoreMesh(
        core_axis_name="c", subcore_axis_name="s", num_cores=NC, num_subcores=NS
    )

    @functools.partial(
        pl.kernel,
        out_type=jax.ShapeDtypeStruct((l, D // 8, b // 128, 8, 128), jnp.float32),
        mesh=mesh,
        compiler_params=pltpu.CompilerParams(
            needs_layout_passes=False, use_tc_tiling_on_sc=False
        ),
        scratch_types=[
            pltpu.VMEM((2, S * l), jnp.int32),
            pltpu.VMEM((2, S * l, D), jnp.float32),
            pltpu.VMEM((2, l, D // 8, 8, S), jnp.float32),
            pltpu.SemaphoreType.DMA,
            pltpu.SemaphoreType.DMA,
            pltpu.SemaphoreType.DMA,
            pltpu.SemaphoreType.DMA,
        ],
    )
    def gather_norm(
        x_hbm, table_hbm, out_hbm, idx_v, rows_v, blk_v, g0, g1, w0, w1
    ):
        wid = lax.axis_index("s") * NC + lax.axis_index("c")
        base = wid * rows_per_w
        lane = lax.iota(jnp.int32, L)
        gsems = (g0, g1)
        wsems = (w0, w1)

        def fire(t, p):
            row0 = base + t * S
            pltpu.sync_copy(x_hbm.at[pl.ds(row0 * l, S * l)], idx_v.at[p])
            for i in range(NDESC):
                pltpu.async_copy(
                    table_hbm.at[idx_v.at[p, pl.ds(i * DESC, DESC)]],
                    rows_v.at[p, pl.ds(i * DESC, DESC)], gsems[p],
                )

        def drain_gather(t, p):
            for i in range(NDESC):
                pltpu.make_async_copy(
                    table_hbm.at[idx_v.at[p, pl.ds(i * DESC, DESC)]],
                    rows_v.at[p, pl.ds(i * DESC, DESC)], gsems[p],
                ).wait()

        dd = [jnp.bitwise_and(jnp.int32(d) + lane, D - 1) for d in range(D)]
        dt_c = [lax.shift_right_logical(v, 3) for v in dd]
        dm_c = [jnp.bitwise_and(v, 7) for v in dd]

        def norm(t, p):
            rows_p = rows_v.at[p]
            blk_p = blk_v.at[p]
            lane_l = lane * l

            def one(j):
                rv = lane_l + j
                jv = jnp.full((L,), j, jnp.int32)
                cols = [plsc.load_gather(rows_p, [rv, dd[d]]) for d in range(D)]
                sq = [c * c for c in cols]
                while len(sq) > 1:
                    sq = [a + b for a, b in zip(sq[::2], sq[1::2])]
                y = _rsqrt_nr(sq[0])
                for d, c in enumerate(cols):
                    plsc.store_scatter(blk_p, [jv, dt_c[d], dm_c[d], lane], c * y)

            def body(j2, carry):
                one(j2 * 2)
                one(j2 * 2 + 1)
                return carry

            lax.fori_loop(0, l // 2, body, 0)

        def _write_dst(t):
            row0 = base + t * S
            return out_hbm.at[:, :, row0 // 128, :, pl.ds(row0 % 128, S)]

        def write(t, p):
            pltpu.async_copy(blk_v.at[p], _write_dst(t), wsems[p])

        def drain_write(t, p):
            pltpu.make_async_copy(blk_v.at[p], _write_dst(t), wsems[p]).wait()

        fire(0, 0)

        def pair(t2, carry):
            for p in (0, 1):
                t = t2 * 2 + p
                drain_gather(t, p)

                @pl.when(t + 1 < steps)
                def _():
                    fire(t + 1, 1 - p)

                @pl.when(t >= 2)
                def _():
                    drain_write(t - 2, p)

                if True:
                    norm(t, p)
                write(t, p)
            return carry

        lax.fori_loop(0, steps // 2, pair, 0)
        drain_write(steps - 2, 0)
        drain_write(steps - 1, 1)

    return gather_norm


def kernel(x, table):
    b, l = x.shape
    n, d = table.shape
    xf, tf = lax.optimization_barrier(
        (x.astype(jnp.int32).reshape(-1), table.reshape(n * d // 128, 128))
    )
    out5 = _build(b, l)(xf, tf.reshape(n, d))
    out_t = out5.transpose(0, 1, 3, 2, 4).reshape(l, D, b)
    return out_t.transpose(2, 0, 1)
